# baseline (speedup 1.0000x reference)
import jax
import jax.numpy as jnp
from jax import lax
from jax.experimental import pallas as pl
from jax.experimental.pallas import tpu as pltpu
from jax.experimental.pallas import tpu_sc as plsc

N_KEYS = 1_000_000
D = 64
NQ = 64
K = 32

CHUNK = 8192
GSIZE = 256
GPC = CHUNK // GSIZE
NCHUNKS = 128
NCHUNKS_REAL = (N_KEYS + CHUNK - 1) // CHUNK
NG = NCHUNKS * GPC

NEG_INF = float("-inf")
BIG_I32 = 2 ** 30


def _score_kernel(q_ref, k_ref, s_ref, gmax_ref):
    c = pl.program_id(0)
    q = q_ref[...]
    qn = q / (jnp.sqrt(jnp.sum(q * q, axis=1, keepdims=True)) + 1e-10)
    kc = k_ref[...]
    sk = jnp.sum(kc * kc, axis=1)
    inv = 1.0 / (jnp.sqrt(sk) + 1e-10)
    s = lax.dot_general(qn, kc, (((1,), (1,)), ((), ())),
                        preferred_element_type=jnp.float32)
    s = s * inv[None, :]
    col = c * CHUNK + lax.broadcasted_iota(jnp.int32, (1, CHUNK), 1)
    s = jnp.where(col < N_KEYS, s, NEG_INF)
    s3 = s.reshape(NQ, GPC, GSIZE)
    s_ref[...] = s3
    gmax_ref[...] = jnp.max(s3, axis=-1).reshape(NQ, 1, 1, GPC)


def _phase_a(queries, keys):
    return pl.pallas_call(
        _score_kernel,
        grid=(NCHUNKS,),
        in_specs=[
            pl.BlockSpec((NQ, D), lambda c: (0, 0)),
            pl.BlockSpec((CHUNK, D),
                         lambda c: (jnp.minimum(c, NCHUNKS_REAL - 1), 0)),
        ],
        out_specs=[
            pl.BlockSpec((NQ, GPC, GSIZE), lambda c: (0, c, 0)),
            pl.BlockSpec((NQ, 1, 1, GPC), lambda c: (0, c, 0, 0)),
        ],
        out_shape=[
            jax.ShapeDtypeStruct((NQ, NG, GSIZE), jnp.float32),
            jax.ShapeDtypeStruct((NQ, NCHUNKS, 1, GPC), jnp.float32),
        ],
        compiler_params=pltpu.CompilerParams(
            dimension_semantics=("arbitrary",),
        ),
    )(queries, keys)


def _iota16():
    return lax.iota(jnp.int32, 16)


def _fsplat(x):
    return jnp.full((16,), x, dtype=jnp.float32)


def _isplat(x):
    return jnp.full((16,), x, dtype=jnp.int32)


def _lane0():
    return _iota16() == 0


def _extract_topk(ref2d, m1_ref, nblk, row_of):
    iota = _iota16()

    def one_iter(it, carry):
        v0, v1, g0, g1 = carry

        def mx(b, acc):
            return jnp.maximum(acc, m1_ref[pl.ds(b * 16, 16)])
        m = jnp.max(lax.fori_loop(0, nblk, mx, _fsplat(NEG_INF)))

        def fnd(b, best):
            v = m1_ref[pl.ds(b * 16, 16)]
            ids = iota + b * 16
            return jnp.minimum(best, jnp.where(v == m, ids, BIG_I32))
        i1 = jnp.min(lax.fori_loop(0, nblk, fnd, _isplat(BIG_I32)))
        bb = i1 // 16
        l = i1 % 16

        rows = row_of(bb, iota // 8)
        cols = (iota % 8) * 16 + l
        col = plsc.load_gather(ref2d, [rows, cols])
        jsel = jnp.min(jnp.where(col == m, iota, _isplat(BIG_I32)))
        g = bb * 256 + jsel * 16 + l

        v0 = jnp.where(iota == it, m, v0)
        v1 = jnp.where(iota == (it - 16), m, v1)
        g0 = jnp.where(iota == it, g, g0)
        g1 = jnp.where(iota == (it - 16), g, g1)

        e = g % 256
        plsc.store_scatter(ref2d, [_isplat(row_of(bb, e // 128)),
                                   _isplat(e % 128)],
                           _fsplat(NEG_INF), mask=_lane0())
        col2 = plsc.load_gather(ref2d, [rows, cols])
        plsc.store_scatter(m1_ref, [_isplat(i1)], _fsplat(jnp.max(col2)),
                           mask=_lane0())
        return v0, v1, g0, g1

    init = (_fsplat(NEG_INF), _fsplat(NEG_INF), _isplat(0), _isplat(0))
    return lax.fori_loop(0, K, one_iter, init)


def _build_m1(ref2d, m1_ref, nblk, row_of):
    def per_block(b, _):
        def mx(j, acc):
            v = ref2d[row_of(b, j // 8), pl.ds((j % 8) * 16, 16)]
            return jnp.maximum(acc, v)
        m1_ref[pl.ds(b * 16, 16)] = lax.fori_loop(0, 16, mx, _fsplat(NEG_INF))
        return 0
    lax.fori_loop(0, nblk, per_block, 0)


def _select_body(scores_hbm, gmax_hbm, out_hbm,
                 gm_ref, m1_ref, cand_ref, m1b_ref, gidx_ref, outv_ref, sem):
    wid = lax.axis_index("s") * 2 + lax.axis_index("c")

    def gm_row(bb, h):
        return 2 * bb + h

    def cand_row(bb, h):
        return bb + 32 * h

    def per_query(t, _):
        q = 2 * wid + t

        pltpu.sync_copy(gmax_hbm.at[pl.ds(q * (NG // 128), NG // 128)], gm_ref)
        _build_m1(gm_ref, m1_ref, NG // 256, gm_row)
        _, _, grp0, grp1 = _extract_topk(gm_ref, m1_ref, NG // 256, gm_row)

        base = 2 * (q * NG)
        gidx_ref[pl.ds(0, 16)] = 2 * grp0 + base
        gidx_ref[pl.ds(16, 16)] = 2 * grp1 + base
        gidx_ref[pl.ds(32, 16)] = 2 * grp0 + (base + 1)
        gidx_ref[pl.ds(48, 16)] = 2 * grp1 + (base + 1)
        pltpu.async_copy(scores_hbm.at[gidx_ref], cand_ref, sem).wait()

        _build_m1(cand_ref, m1b_ref, K, cand_row)
        val0, val1, _, _ = _extract_topk(cand_ref, m1b_ref, K, cand_row)

        outv_ref[pl.ds(t * 32, 16)] = val0
        outv_ref[pl.ds(t * 32 + 16, 16)] = val1
        return 0

    lax.fori_loop(0, 2, per_query, 0)
    pltpu.sync_copy(outv_ref,
                    out_hbm.at[wid // 2, pl.ds((wid % 2) * 64, 64)])


def _phase_b(scores2d, gmax2d):
    mesh = plsc.VectorSubcoreMesh(core_axis_name="c", subcore_axis_name="s",
                                  num_cores=2, num_subcores=16)
    fn = pl.kernel(
        _select_body,
        out_type=jax.ShapeDtypeStruct((NQ * K // 128, 128), jnp.float32),
        mesh=mesh,
        scratch_types=[
            pltpu.VMEM((NG // 128, 128), jnp.float32),
            pltpu.VMEM((NG // 256 * 16,), jnp.float32),
            pltpu.VMEM((2 * K, 128), jnp.float32),
            pltpu.VMEM((K * 16,), jnp.float32),
            pltpu.VMEM((2 * K,), jnp.int32),
            pltpu.VMEM((2 * K,), jnp.float32),
            pltpu.SemaphoreType.DMA,
        ],
        compiler_params=pltpu.CompilerParams(needs_layout_passes=False),
    )
    return fn(scores2d, gmax2d)


def kernel(queries, keys, k):
    scores3, gmax4 = _phase_a(queries, keys)
    scores2d = scores3.reshape(NQ * NG * 2, 128)
    gmax2d = gmax4.reshape(NQ * NG // 128, 128)
    out2d = _phase_b(scores2d, gmax2d)
    return out2d.reshape(NQ, K)
